# baseline (speedup 1.0000x reference)
import functools

import jax
import jax.numpy as jnp
from jax import lax
from jax.experimental import pallas as pl
from jax.experimental.pallas import tpu as pltpu
from jax.experimental.pallas import tpu_sc as plsc
from jax.scipy.linalg import block_diag

_HIGH = jax.lax.Precision.HIGHEST

NC = 2
NS = 16
NW = NC * NS
CH = 128


def _round_up(x, m):
    return (x + m - 1) // m * m


def _dot(a, b):
    return jnp.dot(a, b, preferred_element_type=jnp.float32, precision=_HIGH)


def _precompute_body(x4_ref, bda_ref, bdb_ref, b1t_ref, pa_ref, pb_ref):
    x4 = x4_ref[...]
    pa_ref[...] = _dot(x4, bda_ref[...])
    pb_ref[...] = _dot(x4, bdb_ref[...]) + b1t_ref[...]


def _precompute(x4, bda, bdb, b1t):
    n4 = x4.shape[0]
    return pl.pallas_call(
        _precompute_body,
        out_shape=[jax.ShapeDtypeStruct((n4, 128), jnp.float32),
                   jax.ShapeDtypeStruct((n4, 128), jnp.float32)],
    )(x4, bda, bdb, b1t)


G = 512


def _gather_body(ept0, ept1, eh, pa_hbm, pb_hbm, src_hbm, dst_hbm,
                 ga_hbm, gb_hbm, sidx_v, didx_v, bga0, bga1, bgb0, bgb1,
                 sem_i, sem_g, sem_w0, sem_w1):
    c = lax.axis_index("c")
    s = lax.axis_index("s")
    ept = jnp.where(c == 0, ept0, ept1)
    tbase = jnp.where(c == 0, s * ept0, NS * ept0 + s * ept1)
    ng = ept // G

    emax = max(ept0, ept1)
    h0 = pltpu.async_copy(src_hbm.at[pl.ds(tbase, emax)], sidx_v, sem_i)
    h1 = pltpu.async_copy(dst_hbm.at[pl.ds(tbase, emax)], didx_v, sem_i)
    h0.wait()
    h1.wait()

    def do_group(j, bga, bgb, sem_w, drain_wb):
        base = tbase + j * G
        loc = j * G
        if drain_wb:
            pltpu.make_async_copy(bga, ga_hbm.at[pl.ds(base, G)], sem_w).wait()
            pltpu.make_async_copy(bgb, gb_hbm.at[pl.ds(base, G)], sem_w).wait()
        hs = []
        for k in range(G // CH):
            isl = pl.ds(loc + k * CH, CH)
            bsl = pl.ds(k * CH, CH)
            hs.append(pltpu.async_copy(pa_hbm.at[sidx_v.at[isl]],
                                       bga.at[bsl], sem_g))
            hs.append(pltpu.async_copy(pb_hbm.at[didx_v.at[isl]],
                                       bgb.at[bsl], sem_g))
        for h in hs:
            h.wait()
        pltpu.async_copy(bga, ga_hbm.at[pl.ds(base, G)], sem_w)
        pltpu.async_copy(bgb, gb_hbm.at[pl.ds(base, G)], sem_w)

    do_group(0, bga0, bgb0, sem_w0, False)
    do_group(1, bga1, bgb1, sem_w1, False)

    def body(i, _):
        do_group(2 * i, bga0, bgb0, sem_w0, True)
        do_group(2 * i + 1, bga1, bgb1, sem_w1, True)
        return 0

    lax.fori_loop(1, ng // 2, body, 0)

    for sem_w, bga, bgb in ((sem_w0, bga0, bgb0), (sem_w1, bga1, bgb1)):
        pltpu.make_async_copy(bga, ga_hbm.at[pl.ds(tbase, G)], sem_w).wait()
        pltpu.make_async_copy(bgb, gb_hbm.at[pl.ds(tbase, G)], sem_w).wait()


def _gather(pa, pb, src_p, dst_p, ep, eh, ept0, ept1):
    assert ept0 % (2 * G) == 0 and ept1 % (2 * G) == 0
    assert NS * (ept0 + ept1) == ep
    emax = max(ept0, ept1)
    mesh = plsc.VectorSubcoreMesh(core_axis_name="c", subcore_axis_name="s",
                                  num_cores=NC, num_subcores=NS)
    k = pl.kernel(
        functools.partial(_gather_body, ept0, ept1, eh),
        out_type=[jax.ShapeDtypeStruct((ep, eh), jnp.float32),
                  jax.ShapeDtypeStruct((ep, eh), jnp.float32)],
        mesh=mesh,
        compiler_params=pltpu.CompilerParams(use_tc_tiling_on_sc=False),
        scratch_types=[pltpu.VMEM((emax,), jnp.int32),
                       pltpu.VMEM((emax,), jnp.int32),
                       pltpu.VMEM((G, eh), jnp.float32),
                       pltpu.VMEM((G, eh), jnp.float32),
                       pltpu.VMEM((G, eh), jnp.float32),
                       pltpu.VMEM((G, eh), jnp.float32),
                       pltpu.SemaphoreType.DMA,
                       pltpu.SemaphoreType.DMA,
                       pltpu.SemaphoreType.DMA,
                       pltpu.SemaphoreType.DMA],
    )
    return k(pa, pb, src_p, dst_p)


def _edge_mlp_body(ga_ref, gb_ref, wc_ref, w2bd_ref, b2t_ref, h2_ref):
    h1 = jax.nn.sigmoid(ga_ref[...] + gb_ref[...] + wc_ref[...])
    h2_ref[...] = jax.nn.sigmoid(_dot(h1, w2bd_ref[...]) + b2t_ref[...])


def _edge_mlp(ga128, gb128, wc128, w2bd, b2t, ep4):
    blk = 2048
    grid = (ep4 // blk,)
    full = lambda i: (0, 0)
    return pl.pallas_call(
        _edge_mlp_body,
        grid=grid,
        in_specs=[
            pl.BlockSpec((blk, 128), lambda i: (i, 0)),
            pl.BlockSpec((blk, 128), lambda i: (i, 0)),
            pl.BlockSpec((blk, 128), lambda i: (i, 0)),
            pl.BlockSpec((128, 128), full),
            pl.BlockSpec((1, 128), full),
        ],
        out_specs=pl.BlockSpec((blk, 128), lambda i: (i, 0)),
        out_shape=jax.ShapeDtypeStruct((ep4, 128), jnp.float32),
    )(ga128, gb128, wc128, w2bd, b2t)


def _scatter_body(ept, np_rows, eh, h2_hbm, dst_hbm, src_hbm, zer_hbm,
                  acc_hbm, didx_v, sidx_v, bd0, bd1, obuf_v, acc_d, acc_s,
                  sem_i, sem_a, sem_d0, sem_d1):
    c = lax.axis_index("c")
    s = lax.axis_index("s")
    wid = s * NC + c
    rpt = np_rows // NS
    ng = ept // G
    npt = ept // CH
    tbase = wid * ept

    h0 = pltpu.async_copy(dst_hbm.at[pl.ds(wid * npt, npt)], didx_v, sem_i)
    h1 = pltpu.async_copy(src_hbm.at[pl.ds(wid * npt, npt)], sidx_v, sem_i)

    pltpu.sync_copy(zer_hbm, obuf_v)
    pltpu.sync_copy(obuf_v, acc_d.at[pl.ds(s * rpt, rpt)])
    pltpu.sync_copy(obuf_v, acc_s.at[pl.ds(s * rpt, rpt)])
    h0.wait()
    h1.wait()
    plsc.subcore_barrier()

    pltpu.async_copy(h2_hbm.at[pl.ds(tbase, G)], bd0, sem_d0)
    pltpu.async_copy(h2_hbm.at[pl.ds(tbase + G, G)], bd1, sem_d1)

    def do_group(j, bd, sem_d):
        base = tbase + j * G
        pltpu.make_async_copy(h2_hbm.at[pl.ds(base, G)], bd, sem_d).wait()
        hs = []
        for k in range(G // CH):
            q = j * (G // CH) + k
            bsl = pl.ds(k * CH, CH)
            hs.append(pltpu.async_copy(bd.at[bsl], acc_d.at[didx_v.at[q]],
                                       sem_a, add=True))
            hs.append(pltpu.async_copy(bd.at[bsl], acc_s.at[sidx_v.at[q]],
                                       sem_a, add=True))
        for h in hs:
            h.wait()
        nxt = lax.rem(j + 2, ng)
        pltpu.async_copy(h2_hbm.at[pl.ds(tbase + nxt * G, G)], bd, sem_d)

    def body(i, _):
        do_group(2 * i, bd0, sem_d0)
        do_group(2 * i + 1, bd1, sem_d1)
        return 0

    lax.fori_loop(0, ng // 2, body, 0)

    pltpu.make_async_copy(h2_hbm.at[pl.ds(tbase, G)], bd0, sem_d0).wait()
    pltpu.make_async_copy(h2_hbm.at[pl.ds(tbase, G)], bd1, sem_d1).wait()
    plsc.subcore_barrier()

    pltpu.sync_copy(acc_d.at[pl.ds(s * rpt, rpt)], obuf_v)
    pltpu.sync_copy(obuf_v, acc_hbm.at[pl.ds(c * np_rows + s * rpt, rpt)])
    pltpu.sync_copy(acc_s.at[pl.ds(s * rpt, rpt)], obuf_v)
    pltpu.sync_copy(obuf_v,
                    acc_hbm.at[pl.ds((NC + c) * np_rows + s * rpt, rpt)])


def _scatter(h2, dst2d, src2d, zer, ep, np_rows, eh):
    ept = ep // NW
    assert ept % (2 * G) == 0
    npt = ept // CH
    rpt = np_rows // NS
    mesh = plsc.VectorSubcoreMesh(core_axis_name="c", subcore_axis_name="s",
                                  num_cores=NC, num_subcores=NS)
    k = pl.kernel(
        functools.partial(_scatter_body, ept, np_rows, eh),
        out_type=jax.ShapeDtypeStruct((2 * NC * np_rows, eh), jnp.float32),
        mesh=mesh,
        compiler_params=pltpu.CompilerParams(use_tc_tiling_on_sc=False),
        scratch_types=[pltpu.VMEM((npt, CH), jnp.int32),
                       pltpu.VMEM((npt, CH), jnp.int32),
                       pltpu.VMEM((G, eh), jnp.float32),
                       pltpu.VMEM((G, eh), jnp.float32),
                       pltpu.VMEM((rpt, eh), jnp.float32),
                       pltpu.VMEM_SHARED((np_rows, eh), jnp.float32),
                       pltpu.VMEM_SHARED((np_rows, eh), jnp.float32),
                       pltpu.SemaphoreType.DMA,
                       pltpu.SemaphoreType.DMA,
                       pltpu.SemaphoreType.DMA,
                       pltpu.SemaphoreType.DMA],
    )
    return k(h2, dst2d, src2d, zer)


def _node_mlp_body(n4, npv, acc_ref, w3bd_ref, b3t_ref, out_ref):
    a = (acc_ref[0:n4, :] + acc_ref[npv:npv + n4, :]
         - acc_ref[2 * npv:2 * npv + n4, :]
         - acc_ref[3 * npv:3 * npv + n4, :])
    out_ref[...] = jax.nn.sigmoid(_dot(a, w3bd_ref[...]) + b3t_ref[...])


def _node_mlp(acc128, w3bd, b3t, n4, npv, n_out4):
    return pl.pallas_call(
        functools.partial(_node_mlp_body, n4, npv),
        out_shape=jax.ShapeDtypeStruct((n4, n_out4), jnp.float32),
    )(acc128, w3bd, b3t)


def kernel(node_features, edge_weight, edge_index, W1, b1, W2, b2, W3, b3):
    _, n, d = node_features.shape
    e = edge_index.shape[1]
    eh = W1.shape[1]
    e_out = W2.shape[1]
    n_out = W3.shape[1]

    ep = _round_up(e, NW * CH)
    np_rows = _round_up(n + 1, NS * 8)

    src = edge_index[0]
    dst = edge_index[1]
    w = edge_weight[0]

    ept0 = 4 * (ep // NS) // 10 // (2 * G) * (2 * G)
    ept1 = ep // NS - ept0

    pad = ep - e
    slack = abs(ept0 - ept1)
    gsrc = jnp.pad(src, (0, pad + slack))
    gdst = jnp.pad(dst, (0, pad + slack))
    ssrc = jnp.pad(src, (0, pad), constant_values=n)
    sdst = jnp.pad(dst, (0, pad), constant_values=n)
    wp = jnp.pad(w, (0, pad))

    w1a = W1[:d]
    w1b = W1[d:2 * d]
    c = W1[2 * d]
    w2p = jnp.pad(W2, ((0, 0), (0, eh - e_out)))
    b2p = jnp.pad(b2, (0, eh - e_out))
    w3p = jnp.pad(W3, ((0, eh - e_out), (0, 0)))

    x4 = jnp.reshape(node_features, (n // 4, 4 * d))
    bda = block_diag(w1a, w1a, w1a, w1a)
    bdb = block_diag(w1b, w1b, w1b, w1b)
    b1t = jnp.tile(b1, 4)[None, :]
    w2bd = block_diag(w2p, w2p, w2p, w2p)
    b2t = jnp.tile(b2p, 4)[None, :]
    w3bd = block_diag(w3p, w3p, w3p, w3p)
    b3t = jnp.tile(b3, 4)[None, :]
    c_row = c[None, :]
    sel_c = block_diag(c_row, c_row, c_row, c_row)
    wc128 = jnp.dot(jnp.reshape(wp, (ep // 4, 4)), sel_c,
                    preferred_element_type=jnp.float32)

    zer = jnp.zeros((np_rows // NS, eh), jnp.float32)

    pa128, pb128 = _precompute(x4, bda, bdb, b1t)
    pa = jnp.reshape(pa128, (n, eh))
    pb = jnp.reshape(pb128, (n, eh))
    ga, gb = _gather(pa, pb, gsrc, gdst, ep, eh, ept0, ept1)
    ga128 = jnp.reshape(ga, (ep // 4, 128))
    gb128 = jnp.reshape(gb, (ep // 4, 128))
    h2_128 = _edge_mlp(ga128, gb128, wc128, w2bd, b2t, ep // 4)
    h2 = jnp.reshape(h2_128, (ep, eh))
    sdst2d = jnp.reshape(sdst, (ep // CH, CH))
    ssrc2d = jnp.reshape(ssrc, (ep // CH, CH))
    acc = _scatter(h2, sdst2d, ssrc2d, zer, ep, np_rows, eh)
    acc128 = jnp.reshape(acc, (2 * NC * np_rows // 4, 128))
    out4 = _node_mlp(acc128, w3bd, b3t, n // 4, np_rows // 4, 4 * n_out)
    return jnp.reshape(out4, (1, n, n_out))


# speedup vs baseline: 1.0403x; 1.0403x over previous
import functools

import jax
import jax.numpy as jnp
from jax import lax
from jax.experimental import pallas as pl
from jax.experimental.pallas import tpu as pltpu
from jax.experimental.pallas import tpu_sc as plsc
from jax.scipy.linalg import block_diag

_HIGH = jax.lax.Precision.HIGHEST

NC = 2
NS = 16
NW = NC * NS
CH = 128


def _round_up(x, m):
    return (x + m - 1) // m * m


def _dot(a, b):
    return jnp.dot(a, b, preferred_element_type=jnp.float32, precision=_HIGH)


def _precompute_body(x4_ref, bda_ref, bdb_ref, b1t_ref, pa_ref, pb_ref):
    x4 = x4_ref[...]
    pa_ref[...] = _dot(x4, bda_ref[...])
    pb_ref[...] = _dot(x4, bdb_ref[...]) + b1t_ref[...]


def _precompute(x4, bda, bdb, b1t):
    n4 = x4.shape[0]
    return pl.pallas_call(
        _precompute_body,
        out_shape=[jax.ShapeDtypeStruct((n4, 128), jnp.float32),
                   jax.ShapeDtypeStruct((n4, 128), jnp.float32)],
    )(x4, bda, bdb, b1t)


G = 512


def _gather_body(ept0, ept1, eh, pa_hbm, pb_hbm, src_hbm, dst_hbm,
                 ga_hbm, gb_hbm, sidx_v, didx_v, bga0, bga1, bgb0, bgb1,
                 sem_i, sem_g, sem_w0, sem_w1):
    c = lax.axis_index("c")
    s = lax.axis_index("s")
    ept = jnp.where(c == 0, ept0, ept1)
    tbase = jnp.where(c == 0, s * ept0, NS * ept0 + s * ept1)
    ng = ept // G

    emax = max(ept0, ept1)
    h0 = pltpu.async_copy(src_hbm.at[pl.ds(tbase, emax)], sidx_v, sem_i)
    h1 = pltpu.async_copy(dst_hbm.at[pl.ds(tbase, emax)], didx_v, sem_i)
    h0.wait()
    h1.wait()

    def do_group(j, bga, bgb, sem_w, drain_wb):
        base = tbase + j * G
        loc = j * G
        if drain_wb:
            pltpu.make_async_copy(bga, ga_hbm.at[pl.ds(base, G)], sem_w).wait()
            pltpu.make_async_copy(bgb, gb_hbm.at[pl.ds(base, G)], sem_w).wait()
        hs = []
        for k in range(G // CH):
            isl = pl.ds(loc + k * CH, CH)
            bsl = pl.ds(k * CH, CH)
            hs.append(pltpu.async_copy(pa_hbm.at[sidx_v.at[isl]],
                                       bga.at[bsl], sem_g))
            hs.append(pltpu.async_copy(pb_hbm.at[didx_v.at[isl]],
                                       bgb.at[bsl], sem_g))
        for h in hs:
            h.wait()
        pltpu.async_copy(bga, ga_hbm.at[pl.ds(base, G)], sem_w)
        pltpu.async_copy(bgb, gb_hbm.at[pl.ds(base, G)], sem_w)

    do_group(0, bga0, bgb0, sem_w0, False)
    do_group(1, bga1, bgb1, sem_w1, False)

    def body(i, _):
        do_group(2 * i, bga0, bgb0, sem_w0, True)
        do_group(2 * i + 1, bga1, bgb1, sem_w1, True)
        return 0

    lax.fori_loop(1, ng // 2, body, 0)

    for sem_w, bga, bgb in ((sem_w0, bga0, bgb0), (sem_w1, bga1, bgb1)):
        pltpu.make_async_copy(bga, ga_hbm.at[pl.ds(tbase, G)], sem_w).wait()
        pltpu.make_async_copy(bgb, gb_hbm.at[pl.ds(tbase, G)], sem_w).wait()


def _gather(pa, pb, src_p, dst_p, ep, eh, ept0, ept1):
    assert ept0 % (2 * G) == 0 and ept1 % (2 * G) == 0
    assert NS * (ept0 + ept1) == ep
    emax = max(ept0, ept1)
    mesh = plsc.VectorSubcoreMesh(core_axis_name="c", subcore_axis_name="s",
                                  num_cores=NC, num_subcores=NS)
    k = pl.kernel(
        functools.partial(_gather_body, ept0, ept1, eh),
        out_type=[jax.ShapeDtypeStruct((ep, eh), jnp.float32),
                  jax.ShapeDtypeStruct((ep, eh), jnp.float32)],
        mesh=mesh,
        compiler_params=pltpu.CompilerParams(use_tc_tiling_on_sc=False),
        scratch_types=[pltpu.VMEM((emax,), jnp.int32),
                       pltpu.VMEM((emax,), jnp.int32),
                       pltpu.VMEM((G, eh), jnp.float32),
                       pltpu.VMEM((G, eh), jnp.float32),
                       pltpu.VMEM((G, eh), jnp.float32),
                       pltpu.VMEM((G, eh), jnp.float32),
                       pltpu.SemaphoreType.DMA,
                       pltpu.SemaphoreType.DMA,
                       pltpu.SemaphoreType.DMA,
                       pltpu.SemaphoreType.DMA],
    )
    return k(pa, pb, src_p, dst_p)


def _edge_mlp_body(ga_ref, gb_ref, wc_ref, w2bd_ref, b2t_ref, h2_ref):
    h1 = jax.nn.sigmoid(ga_ref[...] + gb_ref[...] + wc_ref[...])
    h2_ref[...] = jax.nn.sigmoid(_dot(h1, w2bd_ref[...]) + b2t_ref[...])


def _edge_mlp(ga128, gb128, wc128, w2bd, b2t, ep4):
    blk = 2048
    grid = (ep4 // blk,)
    full = lambda i: (0, 0)
    return pl.pallas_call(
        _edge_mlp_body,
        grid=grid,
        in_specs=[
            pl.BlockSpec((blk, 128), lambda i: (i, 0)),
            pl.BlockSpec((blk, 128), lambda i: (i, 0)),
            pl.BlockSpec((blk, 128), lambda i: (i, 0)),
            pl.BlockSpec((128, 128), full),
            pl.BlockSpec((1, 128), full),
        ],
        out_specs=pl.BlockSpec((blk, 128), lambda i: (i, 0)),
        out_shape=jax.ShapeDtypeStruct((ep4, 128), jnp.float32),
    )(ga128, gb128, wc128, w2bd, b2t)


def _scatter_body(ept, np_rows, eh, h2_hbm, dst_hbm, src_hbm, zer_hbm,
                  acc_hbm, didx_v, sidx_v, bd0, bd1, obuf_v, acc_d, acc_s,
                  sem_i, sem_a, sem_d0, sem_d1):
    c = lax.axis_index("c")
    s = lax.axis_index("s")
    wid = s * NC + c
    rpt = np_rows // NS
    ng = ept // G
    npt = ept // CH
    tbase = wid * ept

    h0 = pltpu.async_copy(dst_hbm.at[pl.ds(wid * npt, npt)], didx_v, sem_i)
    h1 = pltpu.async_copy(src_hbm.at[pl.ds(wid * npt, npt)], sidx_v, sem_i)

    pltpu.sync_copy(zer_hbm, obuf_v)
    pltpu.sync_copy(obuf_v, acc_d.at[pl.ds(s * rpt, rpt)])
    pltpu.sync_copy(obuf_v, acc_s.at[pl.ds(s * rpt, rpt)])
    h0.wait()
    h1.wait()
    plsc.subcore_barrier()

    pltpu.async_copy(h2_hbm.at[pl.ds(tbase, G)], bd0, sem_d0)
    pltpu.async_copy(h2_hbm.at[pl.ds(tbase + G, G)], bd1, sem_d1)

    def do_group(j, bd, sem_d):
        base = tbase + j * G
        pltpu.make_async_copy(h2_hbm.at[pl.ds(base, G)], bd, sem_d).wait()
        hs = []
        for k in range(G // CH):
            q = j * (G // CH) + k
            bsl = pl.ds(k * CH, CH)
            hs.append(pltpu.async_copy(bd.at[bsl], acc_d.at[didx_v.at[q]],
                                       sem_a, add=True))
            hs.append(pltpu.async_copy(bd.at[bsl], acc_s.at[sidx_v.at[q]],
                                       sem_a, add=True))
        for h in hs:
            h.wait()
        nxt = lax.rem(j + 2, ng)
        pltpu.async_copy(h2_hbm.at[pl.ds(tbase + nxt * G, G)], bd, sem_d)

    def body(i, _):
        do_group(2 * i, bd0, sem_d0)
        do_group(2 * i + 1, bd1, sem_d1)
        return 0

    lax.fori_loop(0, ng // 2, body, 0)

    pltpu.make_async_copy(h2_hbm.at[pl.ds(tbase, G)], bd0, sem_d0).wait()
    pltpu.make_async_copy(h2_hbm.at[pl.ds(tbase, G)], bd1, sem_d1).wait()
    plsc.subcore_barrier()

    pltpu.sync_copy(acc_d.at[pl.ds(s * rpt, rpt)], obuf_v)
    pltpu.sync_copy(obuf_v, acc_hbm.at[pl.ds(c * np_rows + s * rpt, rpt)])
    pltpu.sync_copy(acc_s.at[pl.ds(s * rpt, rpt)], obuf_v)
    pltpu.sync_copy(obuf_v,
                    acc_hbm.at[pl.ds((NC + c) * np_rows + s * rpt, rpt)])


def _scatter(h2, dst2d, src2d, zer, ep, np_rows, eh):
    ept = ep // NW
    assert ept % (2 * G) == 0
    npt = ept // CH
    rpt = np_rows // NS
    mesh = plsc.VectorSubcoreMesh(core_axis_name="c", subcore_axis_name="s",
                                  num_cores=NC, num_subcores=NS)
    k = pl.kernel(
        functools.partial(_scatter_body, ept, np_rows, eh),
        out_type=jax.ShapeDtypeStruct((2 * NC * np_rows, eh), jnp.float32),
        mesh=mesh,
        compiler_params=pltpu.CompilerParams(use_tc_tiling_on_sc=False),
        scratch_types=[pltpu.VMEM((npt, CH), jnp.int32),
                       pltpu.VMEM((npt, CH), jnp.int32),
                       pltpu.VMEM((G, eh), jnp.float32),
                       pltpu.VMEM((G, eh), jnp.float32),
                       pltpu.VMEM((rpt, eh), jnp.float32),
                       pltpu.VMEM_SHARED((np_rows, eh), jnp.float32),
                       pltpu.VMEM_SHARED((np_rows, eh), jnp.float32),
                       pltpu.SemaphoreType.DMA,
                       pltpu.SemaphoreType.DMA,
                       pltpu.SemaphoreType.DMA,
                       pltpu.SemaphoreType.DMA],
    )
    return k(h2, dst2d, src2d, zer)


def _node_mlp_body(n4, npv, acc_ref, w3bd_ref, b3t_ref, out_ref):
    a = (acc_ref[0:n4, :] + acc_ref[npv:npv + n4, :]
         - acc_ref[2 * npv:2 * npv + n4, :]
         - acc_ref[3 * npv:3 * npv + n4, :])
    out_ref[...] = jax.nn.sigmoid(_dot(a, w3bd_ref[...]) + b3t_ref[...])


def _node_mlp(acc128, w3bd, b3t, n4, npv, n_out4):
    return pl.pallas_call(
        functools.partial(_node_mlp_body, n4, npv),
        out_shape=jax.ShapeDtypeStruct((n4, n_out4), jnp.float32),
    )(acc128, w3bd, b3t)


def kernel(node_features, edge_weight, edge_index, W1, b1, W2, b2, W3, b3):
    _, n, d = node_features.shape
    e = edge_index.shape[1]
    eh = W1.shape[1]
    e_out = W2.shape[1]
    n_out = W3.shape[1]

    ep = _round_up(e, NW * CH)
    np_rows = _round_up(n + 1, NS * 8)

    src = edge_index[0]
    dst = edge_index[1]
    w = edge_weight[0]

    ept0 = (ep // NS) // 2 // (2 * G) * (2 * G)
    ept1 = ep // NS - ept0

    pad = ep - e
    slack = abs(ept0 - ept1)
    gsrc = jnp.pad(src, (0, pad + slack))
    gdst = jnp.pad(dst, (0, pad + slack))
    ssrc = jnp.pad(src, (0, pad), constant_values=n)
    sdst = jnp.pad(dst, (0, pad), constant_values=n)
    wp = jnp.pad(w, (0, pad))

    w1a = W1[:d]
    w1b = W1[d:2 * d]
    c = W1[2 * d]
    w2p = jnp.pad(W2, ((0, 0), (0, eh - e_out)))
    b2p = jnp.pad(b2, (0, eh - e_out))
    w3p = jnp.pad(W3, ((0, eh - e_out), (0, 0)))

    x4 = jnp.reshape(node_features, (n // 4, 4 * d))
    bda = block_diag(w1a, w1a, w1a, w1a)
    bdb = block_diag(w1b, w1b, w1b, w1b)
    b1t = jnp.tile(b1, 4)[None, :]
    w2bd = block_diag(w2p, w2p, w2p, w2p)
    b2t = jnp.tile(b2p, 4)[None, :]
    w3bd = block_diag(w3p, w3p, w3p, w3p)
    b3t = jnp.tile(b3, 4)[None, :]
    c_row = c[None, :]
    sel_c = block_diag(c_row, c_row, c_row, c_row)
    wc128 = jnp.dot(jnp.reshape(wp, (ep // 4, 4)), sel_c,
                    preferred_element_type=jnp.float32)

    zer = jnp.zeros((np_rows // NS, eh), jnp.float32)

    pa128, pb128 = _precompute(x4, bda, bdb, b1t)
    pa = jnp.reshape(pa128, (n, eh))
    pb = jnp.reshape(pb128, (n, eh))
    ga, gb = _gather(pa, pb, gsrc, gdst, ep, eh, ept0, ept1)
    ga128 = jnp.reshape(ga, (ep // 4, 128))
    gb128 = jnp.reshape(gb, (ep // 4, 128))
    h2_128 = _edge_mlp(ga128, gb128, wc128, w2bd, b2t, ep // 4)
    h2 = jnp.reshape(h2_128, (ep, eh))
    sdst2d = jnp.reshape(sdst, (ep // CH, CH))
    ssrc2d = jnp.reshape(ssrc, (ep // CH, CH))
    acc = _scatter(h2, sdst2d, ssrc2d, zer, ep, np_rows, eh)
    acc128 = jnp.reshape(acc, (2 * NC * np_rows // 4, 128))
    out4 = _node_mlp(acc128, w3bd, b3t, n // 4, np_rows // 4, 4 * n_out)
    return jnp.reshape(out4, (1, n, n_out))
